# Initial kernel scaffold; baseline (speedup 1.0000x reference)
import jax
import jax.numpy as jnp
from jax import lax
from jax.experimental import pallas as pl
from jax.experimental.pallas import tpu as pltpu
from jax.experimental.pallas import tpu_sc as plsc

N_TOKENS = 1000000
D = 32
BATCH = 16384
HIST = 50
B_TOTAL = BATCH * HIST

NC, NS = 2, 16
NW = NC * NS
ROWS_PER_W = B_TOTAL // NW
CHUNK = 1600
NCHUNK = ROWS_PER_W // CHUNK


def _gather_body(x_hbm, w_hbm, out_hbm, idx_v, rows_v, sem):
    wid = lax.axis_index("s") * NC + lax.axis_index("c")
    base = wid * ROWS_PER_W

    def step(c, carry):
        off = base + c * CHUNK
        pltpu.sync_copy(x_hbm.at[pl.ds(off, CHUNK)], idx_v)
        pltpu.async_copy(w_hbm.at[idx_v], rows_v, sem).wait()
        pltpu.sync_copy(rows_v, out_hbm.at[pl.ds(off, CHUNK)])
        return carry

    lax.fori_loop(0, NCHUNK, step, 0)


def kernel(x, weights):
    x_flat = x.reshape(-1).astype(jnp.int32)
    mesh = plsc.VectorSubcoreMesh(core_axis_name="c", subcore_axis_name="s",
                                  num_cores=NC, num_subcores=NS)
    out = pl.kernel(
        _gather_body,
        out_type=jax.ShapeDtypeStruct((B_TOTAL, D), jnp.float32),
        mesh=mesh,
        scratch_types=[
            pltpu.VMEM((CHUNK,), jnp.int32),
            pltpu.VMEM((CHUNK, D), jnp.float32),
            pltpu.SemaphoreType.DMA,
        ],
    )(x_flat, weights)
    return out.reshape(BATCH, HIST, D)


# speedup vs baseline: 1.1025x; 1.1025x over previous
import jax
import jax.numpy as jnp
from jax import lax
from jax.experimental import pallas as pl
from jax.experimental.pallas import tpu as pltpu
from jax.experimental.pallas import tpu_sc as plsc

N_TOKENS = 1000000
D = 32
BATCH = 16384
HIST = 50
B_TOTAL = BATCH * HIST

NC, NS = 2, 16
NW = NC * NS
ROWS_PER_W = B_TOTAL // NW
CHUNK = 1600
NCHUNK = ROWS_PER_W // CHUNK


def _gather_body(x_hbm, w_hbm, out_hbm, idx_v, rows_v, sem):
    wid = lax.axis_index("s") * NC + lax.axis_index("c")
    base = wid * ROWS_PER_W

    def step(c, carry):
        off = base + c * CHUNK
        pltpu.sync_copy(x_hbm.at[pl.ds(off, CHUNK)], idx_v)
        pltpu.async_copy(w_hbm.at[idx_v], rows_v, sem).wait()
        pltpu.sync_copy(rows_v, out_hbm.at[pl.ds(off, CHUNK)])
        return carry

    lax.fori_loop(0, NCHUNK, step, 0)


def kernel(x, weights):
    x_flat = x.reshape(-1).astype(jnp.int32)
    mesh = plsc.VectorSubcoreMesh(core_axis_name="c", subcore_axis_name="s",
                                  num_cores=NC, num_subcores=NS)
    out = pl.kernel(
        _gather_body,
        out_type=jax.ShapeDtypeStruct((B_TOTAL, D), jnp.float32),
        mesh=mesh,
        scratch_types=[
            pltpu.VMEM((CHUNK,), jnp.int32),
            pltpu.VMEM((CHUNK, D), jnp.float32),
            pltpu.SemaphoreType.DMA,
        ],
        compiler_params=pltpu.CompilerParams(use_tc_tiling_on_sc=False),
    )(x_flat, weights)
    return out.reshape(BATCH, HIST, D)


# speedup vs baseline: 1.1095x; 1.0063x over previous
import jax
import jax.numpy as jnp
from jax import lax
from jax.experimental import pallas as pl
from jax.experimental.pallas import tpu as pltpu
from jax.experimental.pallas import tpu_sc as plsc

N_TOKENS = 1000000
D = 32
BATCH = 16384
HIST = 50
B_TOTAL = BATCH * HIST

NC, NS = 2, 16
NW = NC * NS
ROWS_PER_W = B_TOTAL // NW
CHUNK = 1600
NCHUNK = ROWS_PER_W // CHUNK
NBUF = 2


def _gather_body(x_hbm, w_hbm, out_hbm, idx_v, rows_v, sem_g, sem_s):
    wid = lax.axis_index("s") * NC + lax.axis_index("c")
    base = wid * ROWS_PER_W

    def idx_gather_start(c, b):
        off = base + c * CHUNK
        pltpu.sync_copy(x_hbm.at[pl.ds(off, CHUNK)], idx_v.at[b])
        pltpu.async_copy(w_hbm.at[idx_v.at[b]], rows_v.at[b], sem_g)

    def gather_wait(b):
        pltpu.make_async_copy(w_hbm.at[idx_v.at[b]], rows_v.at[b], sem_g).wait()

    def store_start(c, b):
        off = base + c * CHUNK
        pltpu.async_copy(rows_v.at[b], out_hbm.at[pl.ds(off, CHUNK)], sem_s)

    def store_wait(c, b):
        off = base + c * CHUNK
        pltpu.make_async_copy(rows_v.at[b], out_hbm.at[pl.ds(off, CHUNK)],
                              sem_s).wait()

    for b in range(NBUF):
        idx_gather_start(b, b)

    @pl.loop(0, NCHUNK - NBUF, step=NBUF)
    def _steady(c0):
        for b in range(NBUF):
            c = c0 + b
            gather_wait(b)
            store_start(c, b)
            store_wait(c, b)
            idx_gather_start(c + NBUF, b)

    for b in range(NBUF):
        gather_wait(b)
        store_start(NCHUNK - NBUF + b, b)
    for b in range(NBUF):
        store_wait(NCHUNK - NBUF + b, b)


def kernel(x, weights):
    x_flat = x.reshape(-1).astype(jnp.int32)
    mesh = plsc.VectorSubcoreMesh(core_axis_name="c", subcore_axis_name="s",
                                  num_cores=NC, num_subcores=NS)
    out = pl.kernel(
        _gather_body,
        out_type=jax.ShapeDtypeStruct((B_TOTAL, D), jnp.float32),
        mesh=mesh,
        scratch_types=[
            pltpu.VMEM((NBUF, CHUNK), jnp.int32),
            pltpu.VMEM((NBUF, CHUNK, D), jnp.float32),
            pltpu.SemaphoreType.DMA,
            pltpu.SemaphoreType.DMA,
        ],
        compiler_params=pltpu.CompilerParams(use_tc_tiling_on_sc=False),
    )(x_flat, weights)
    return out.reshape(BATCH, HIST, D)
